# Initial kernel scaffold; baseline (speedup 1.0000x reference)
import jax
import jax.numpy as jnp
from jax.experimental import pallas as pl


def kernel(x, edge_index, W_neigh1, W_self1, b1, W_neigh2, W_self2, b2):
    raise NotImplementedError("write your pallas kernel here")


# speedup vs baseline: 4.5283x; 4.5283x over previous
import functools

import jax
import jax.numpy as jnp
from jax import lax
from jax.experimental import pallas as pl
from jax.experimental.pallas import tpu as pltpu
from jax.experimental.pallas import tpu_sc as plsc

N_NODES = 10000
D = 128
NC = 2
NS = 16
NW = NC * NS
K = 128
N_PAD = 10240
ROWS_PER_TILE = N_PAD // NS
ZROWS = 64


def _sc_aggregate(x_pad, src, dst, e_pad, with_deg):
    pw = e_pad // NW
    n_chunks = pw // K
    mesh = plsc.VectorSubcoreMesh(core_axis_name="c", subcore_axis_name="s")

    out_type = [jax.ShapeDtypeStruct((NC, N_PAD, D), jnp.float32)]
    scratch = [
        pltpu.VMEM((K,), jnp.int32),
        pltpu.VMEM((K,), jnp.int32),
        pltpu.VMEM((K, D), jnp.float32),
        pltpu.VMEM((ZROWS, D), jnp.float32),
        pltpu.VMEM_SHARED((N_PAD, D), jnp.float32),
        pltpu.SemaphoreType.DMA,
    ]
    if with_deg:
        out_type.append(jax.ShapeDtypeStruct((NC, N_PAD), jnp.float32))
        scratch += [
            pltpu.VMEM((K,), jnp.float32),
            pltpu.VMEM((ZROWS,), jnp.float32),
            pltpu.VMEM_SHARED((N_PAD,), jnp.float32),
        ]

    @functools.partial(pl.kernel, mesh=mesh, out_type=out_type,
                       scratch_types=scratch)
    def agg(x_hbm, src_hbm, dst_hbm, *refs):
        if with_deg:
            (acc_hbm, deg_hbm, src_v, dst_v, rows_v, zrow_v, acc_sh, sem,
             ones_v, zdeg_v, deg_sh) = refs
        else:
            acc_hbm, src_v, dst_v, rows_v, zrow_v, acc_sh, sem = refs
        c = lax.axis_index("c")
        s = lax.axis_index("s")
        wid = s * NC + c

        zero16 = jnp.zeros((16,), jnp.float32)

        def fill_zrow(i, _):
            r = i // (D // 16)
            col = (i % (D // 16)) * 16
            zrow_v[r, pl.ds(col, 16)] = zero16
            return 0
        lax.fori_loop(0, ZROWS * (D // 16), fill_zrow, 0)

        if with_deg:
            one16 = jnp.ones((16,), jnp.float32)
            def fill_zdeg(i, _):
                zdeg_v[pl.ds(i * 16, 16)] = zero16
                return 0
            lax.fori_loop(0, ZROWS // 16, fill_zdeg, 0)

            def fill_ones(i, _):
                ones_v[pl.ds(i * 16, 16)] = one16
                return 0
            lax.fori_loop(0, K // 16, fill_ones, 0)

        row0 = s * ROWS_PER_TILE
        def zero_acc(i, _):
            pltpu.sync_copy(zrow_v, acc_sh.at[pl.ds(row0 + i * ZROWS, ZROWS)])
            if with_deg:
                pltpu.sync_copy(zdeg_v,
                                deg_sh.at[pl.ds(row0 + i * ZROWS, ZROWS)])
            return 0
        lax.fori_loop(0, ROWS_PER_TILE // ZROWS, zero_acc, 0)

        plsc.subcore_barrier()

        def chunk(i, _):
            base = wid * pw + i * K
            pltpu.sync_copy(src_hbm.at[pl.ds(base, K)], src_v)
            pltpu.sync_copy(dst_hbm.at[pl.ds(base, K)], dst_v)
            pltpu.async_copy(x_hbm.at[src_v], rows_v, sem).wait()
            pltpu.sync_copy(rows_v, acc_sh.at[dst_v], add=True)
            if with_deg:
                pltpu.sync_copy(ones_v, deg_sh.at[dst_v], add=True)
            return 0
        lax.fori_loop(0, n_chunks, chunk, 0)

        plsc.subcore_barrier()

        pltpu.sync_copy(acc_sh.at[pl.ds(row0, ROWS_PER_TILE)],
                        acc_hbm.at[c, pl.ds(row0, ROWS_PER_TILE)])
        if with_deg:
            pltpu.sync_copy(deg_sh.at[pl.ds(row0, ROWS_PER_TILE)],
                            deg_hbm.at[c, pl.ds(row0, ROWS_PER_TILE)])

    res = agg(x_pad, src, dst)
    if not isinstance(res, (list, tuple)):
        res = (res,)
    return tuple(res)


BLK = 512


def _tc_layer(acc, deg, x_pad, wn, ws, b2d, relu):
    def body(acc_ref, deg_ref, x_ref, wn_ref, ws_ref, b_ref, o_ref):
        a = acc_ref[0] + acc_ref[1]
        dg = (deg_ref[0] + deg_ref[1])[:, None]
        mean = a / jnp.maximum(dg, 1.0)
        out = jnp.dot(mean, wn_ref[...], preferred_element_type=jnp.float32)
        out = out + jnp.dot(x_ref[...], ws_ref[...],
                            preferred_element_type=jnp.float32)
        out = out + b_ref[...]
        if relu:
            out = jnp.maximum(out, 0.0)
        o_ref[...] = out

    return pl.pallas_call(
        body,
        grid=(N_PAD // BLK,),
        in_specs=[
            pl.BlockSpec((NC, BLK, D), lambda i: (0, i, 0)),
            pl.BlockSpec((NC, BLK), lambda i: (0, i)),
            pl.BlockSpec((BLK, D), lambda i: (i, 0)),
            pl.BlockSpec((D, D), lambda i: (0, 0)),
            pl.BlockSpec((D, D), lambda i: (0, 0)),
            pl.BlockSpec((1, D), lambda i: (0, 0)),
        ],
        out_specs=pl.BlockSpec((BLK, D), lambda i: (i, 0)),
        out_shape=jax.ShapeDtypeStruct((N_PAD, D), jnp.float32),
    )(acc, deg, x_pad, wn, ws, b2d)


def kernel(x, edge_index, W_neigh1, W_self1, b1, W_neigh2, W_self2, b2):
    src = edge_index[0].astype(jnp.int32)
    dst = edge_index[1].astype(jnp.int32)
    e = src.shape[0]
    e_pad = ((e + NW * K - 1) // (NW * K)) * (NW * K)
    if e_pad > e:
        src = jnp.concatenate([src, jnp.zeros((e_pad - e,), jnp.int32)])
        dst = jnp.concatenate([dst, jnp.full((e_pad - e,), N_NODES, jnp.int32)])

    x_pad = jnp.pad(x, ((0, N_PAD - N_NODES), (0, 0)))

    acc1, deg = _sc_aggregate(x_pad, src, dst, e_pad, with_deg=True)
    h = _tc_layer(acc1, deg, x_pad, W_neigh1, W_self1,
                  b1.reshape(1, D), relu=True)
    (acc2,) = _sc_aggregate(h, src, dst, e_pad, with_deg=False)
    out = _tc_layer(acc2, deg, h, W_neigh2, W_self2,
                    b2.reshape(1, D), relu=False)
    return out[:N_NODES]


# speedup vs baseline: 5.5511x; 1.2259x over previous
import functools

import jax
import jax.numpy as jnp
from jax import lax
from jax.experimental import pallas as pl
from jax.experimental.pallas import tpu as pltpu
from jax.experimental.pallas import tpu_sc as plsc

N_NODES = 10000
D = 128
NC = 2
NS = 16
NW = NC * NS
K = 64
N_PAD = 10240
ROWS_PER_TILE = N_PAD // NS


def _sc_aggregate(x_pad, src, dst, e_pad, with_deg):
    n_chunks = e_pad // (NW * K)
    pw = n_chunks * K
    mesh = plsc.VectorSubcoreMesh(core_axis_name="c", subcore_axis_name="s")

    out_type = [jax.ShapeDtypeStruct((NC, N_PAD, D), jnp.float32)]
    scratch = [
        pltpu.VMEM((K,), jnp.int32),
        pltpu.VMEM((K,), jnp.int32),
        pltpu.VMEM((K,), jnp.int32),
        pltpu.VMEM((K,), jnp.int32),
        pltpu.VMEM((K, D), jnp.float32),
        pltpu.VMEM((K, D), jnp.float32),
        pltpu.VMEM_SHARED((N_PAD, D), jnp.float32),
        pltpu.SemaphoreType.DMA,
        pltpu.SemaphoreType.DMA,
        pltpu.SemaphoreType.DMA,
        pltpu.SemaphoreType.DMA,
    ]
    if with_deg:
        out_type.append(jax.ShapeDtypeStruct((NC, N_PAD), jnp.float32))
        scratch += [
            pltpu.VMEM((K,), jnp.float32),
            pltpu.VMEM((K,), jnp.float32),
            pltpu.VMEM_SHARED((N_PAD,), jnp.float32),
        ]

    @functools.partial(pl.kernel, mesh=mesh, out_type=out_type,
                       scratch_types=scratch)
    def agg(x_hbm, src_hbm, dst_hbm, *refs):
        if with_deg:
            (acc_hbm, deg_hbm, s0, d0, s1, d1, r0, r1, acc_sh,
             semi0, semi1, semr0, semr1, ones_v, zdeg_v, deg_sh) = refs
        else:
            (acc_hbm, s0, d0, s1, d1, r0, r1, acc_sh,
             semi0, semi1, semr0, semr1) = refs
        c = lax.axis_index("c")
        s = lax.axis_index("s")
        wid = s * NC + c
        base = wid * pw

        zero16 = jnp.zeros((16,), jnp.float32)

        def fill_zrow(i, _):
            r = i // (D // 16)
            col = (i % (D // 16)) * 16
            r0[r, pl.ds(col, 16)] = zero16
            return 0
        lax.fori_loop(0, K * (D // 16), fill_zrow, 0)

        if with_deg:
            one16 = jnp.ones((16,), jnp.float32)
            def fill_deg_bufs(i, _):
                zdeg_v[pl.ds(i * 16, 16)] = zero16
                ones_v[pl.ds(i * 16, 16)] = one16
                return 0
            lax.fori_loop(0, K // 16, fill_deg_bufs, 0)

        row0 = s * ROWS_PER_TILE
        def zero_acc(i, _):
            pltpu.sync_copy(r0, acc_sh.at[pl.ds(row0 + i * K, K)])
            if with_deg:
                pltpu.sync_copy(zdeg_v,
                                deg_sh.at[pl.ds(row0 + i * K, K)])
            return 0
        lax.fori_loop(0, ROWS_PER_TILE // K, zero_acc, 0)

        plsc.subcore_barrier()

        def idx_load(g, sv, dv, sem):
            pltpu.async_copy(src_hbm.at[pl.ds(base + g * K, K)], sv, sem)
            pltpu.async_copy(dst_hbm.at[pl.ds(base + g * K, K)], dv, sem)

        def idx_wait(g, sv, dv, sem):
            pltpu.make_async_copy(
                src_hbm.at[pl.ds(base + g * K, K)], sv, sem).wait()
            pltpu.make_async_copy(
                dst_hbm.at[pl.ds(base + g * K, K)], dv, sem).wait()

        def scat(rbuf, dv):
            pltpu.sync_copy(rbuf, acc_sh.at[dv], add=True)
            if with_deg:
                pltpu.sync_copy(ones_v, deg_sh.at[dv], add=True)

        idx_load(0, s0, d0, semi0)
        idx_load(1, s1, d1, semi1)
        idx_wait(0, s0, d0, semi0)
        pltpu.async_copy(x_hbm.at[s0], r0, semr0)

        def body(t, _):
            g = 2 * t
            idx_wait(g + 1, s1, d1, semi1)
            pltpu.async_copy(x_hbm.at[s1], r1, semr1)
            pltpu.make_async_copy(x_hbm.at[s0], r0, semr0).wait()
            scat(r0, d0)
            idx_load(g + 2, s0, d0, semi0)
            pltpu.make_async_copy(x_hbm.at[s1], r1, semr1).wait()
            scat(r1, d1)
            idx_load(g + 3, s1, d1, semi1)
            idx_wait(g + 2, s0, d0, semi0)
            pltpu.async_copy(x_hbm.at[s0], r0, semr0)
            return 0
        lax.fori_loop(0, (n_chunks - 2) // 2, body, 0)

        idx_wait(n_chunks - 1, s1, d1, semi1)
        pltpu.async_copy(x_hbm.at[s1], r1, semr1)
        pltpu.make_async_copy(x_hbm.at[s0], r0, semr0).wait()
        scat(r0, d0)
        pltpu.make_async_copy(x_hbm.at[s1], r1, semr1).wait()
        scat(r1, d1)

        plsc.subcore_barrier()

        pltpu.sync_copy(acc_sh.at[pl.ds(row0, ROWS_PER_TILE)],
                        acc_hbm.at[c, pl.ds(row0, ROWS_PER_TILE)])
        if with_deg:
            pltpu.sync_copy(deg_sh.at[pl.ds(row0, ROWS_PER_TILE)],
                            deg_hbm.at[c, pl.ds(row0, ROWS_PER_TILE)])

    res = agg(x_pad, src, dst)
    if not isinstance(res, (list, tuple)):
        res = (res,)
    return tuple(res)


BLK = 512


def _tc_layer(acc, deg, x_pad, wn, ws, b2d, relu):
    def body(acc_ref, deg_ref, x_ref, wn_ref, ws_ref, b_ref, o_ref):
        a = acc_ref[0] + acc_ref[1]
        dg = (deg_ref[0] + deg_ref[1])[:, None]
        mean = a / jnp.maximum(dg, 1.0)
        out = jnp.dot(mean, wn_ref[...], preferred_element_type=jnp.float32)
        out = out + jnp.dot(x_ref[...], ws_ref[...],
                            preferred_element_type=jnp.float32)
        out = out + b_ref[...]
        if relu:
            out = jnp.maximum(out, 0.0)
        o_ref[...] = out

    return pl.pallas_call(
        body,
        grid=(N_PAD // BLK,),
        in_specs=[
            pl.BlockSpec((NC, BLK, D), lambda i: (0, i, 0)),
            pl.BlockSpec((NC, BLK), lambda i: (0, i)),
            pl.BlockSpec((BLK, D), lambda i: (i, 0)),
            pl.BlockSpec((D, D), lambda i: (0, 0)),
            pl.BlockSpec((D, D), lambda i: (0, 0)),
            pl.BlockSpec((1, D), lambda i: (0, 0)),
        ],
        out_specs=pl.BlockSpec((BLK, D), lambda i: (i, 0)),
        out_shape=jax.ShapeDtypeStruct((N_PAD, D), jnp.float32),
    )(acc, deg, x_pad, wn, ws, b2d)


def kernel(x, edge_index, W_neigh1, W_self1, b1, W_neigh2, W_self2, b2):
    src = edge_index[0].astype(jnp.int32)
    dst = edge_index[1].astype(jnp.int32)
    e = src.shape[0]
    n_chunks = (e + NW * K - 1) // (NW * K)
    n_chunks += n_chunks % 2
    e_pad = n_chunks * NW * K
    if e_pad > e:
        src = jnp.concatenate([src, jnp.zeros((e_pad - e,), jnp.int32)])
        dst = jnp.concatenate([dst, jnp.full((e_pad - e,), N_NODES, jnp.int32)])

    x_pad = jnp.pad(x, ((0, N_PAD - N_NODES), (0, 0)))

    acc1, deg = _sc_aggregate(x_pad, src, dst, e_pad, with_deg=True)
    h = _tc_layer(acc1, deg, x_pad, W_neigh1, W_self1,
                  b1.reshape(1, D), relu=True)
    (acc2,) = _sc_aggregate(h, src, dst, e_pad, with_deg=False)
    out = _tc_layer(acc2, deg, h, W_neigh2, W_self2,
                    b2.reshape(1, D), relu=False)
    return out[:N_NODES]
